# baseline (speedup 1.0000x reference)
import jax
import jax.numpy as jnp
from jax import lax
from jax.experimental import pallas as pl

_EPSILON = 1e-06
_SOFT_MARGIN = 0.5
_HARD_MARGIN = 0.1
_NUM_NEG = 8
_B = 16
_D = 128
_T = 4096


_G = 4
_TB = _T // _G


def _loss_kernel(ctx_ref, prin_ref, actT_ref, rowid_ref, out_ref):
    logits = lax.dot_general(ctx_ref[...], actT_ref[...],
                             dimension_numbers=(((1,), (1,)), ((), ())),
                             preferred_element_type=jnp.float32)
    logits = logits * (1.0 / (_D ** 0.5))
    sc = jax.nn.sigmoid(logits)
    s = -jnp.log(_EPSILON + 1.0 - sc)

    row = lax.broadcasted_iota(jnp.int32, (_B, _TB), 0)
    rid = rowid_ref[...].reshape(1, _TB)
    off = (row - rid) & (_B - 1)

    pos = off == 0
    neg = (off >= 1) & (off <= _NUM_NEG)

    s_pos = jnp.sum(jnp.where(pos, s, 0.0), axis=0, keepdims=True)

    prin_row = jnp.transpose(prin_ref[...])
    prin_q = jnp.sum(jnp.where(pos, prin_row, 0), axis=0, keepdims=True)
    w = jnp.where(neg & (prin_row != prin_q), 1.0, 0.0)

    x = s - s_pos + _HARD_MARGIN
    quad = x * x / (2.0 * _SOFT_MARGIN)
    lin = x - _SOFT_MARGIN / 2.0
    h = jnp.where(x <= 0.0, 0.0, jnp.where(x < _SOFT_MARGIN, quad, lin))

    num = jnp.sum(h * w, axis=0, keepdims=True)
    den = jnp.sum(w, axis=0, keepdims=True) + _EPSILON
    per_query = num / den
    partial = jnp.sum(per_query, axis=1, keepdims=True) * (1.0 / (_T + _EPSILON))

    @pl.when(pl.program_id(0) == 0)
    def _init():
        out_ref[...] = partial

    @pl.when(pl.program_id(0) > 0)
    def _acc():
        out_ref[...] += partial


def kernel(context_embeddings, principals, action_flat, action_rowids):
    principals = jnp.squeeze(principals).reshape(1, _B)
    actT = action_flat
    rowids = action_rowids
    out = pl.pallas_call(
        _loss_kernel,
        grid=(_G,),
        in_specs=[
            pl.BlockSpec((_B, _D), lambda i: (0, 0)),
            pl.BlockSpec((1, _B), lambda i: (0, 0)),
            pl.BlockSpec((_TB, _D), lambda i: (i, 0)),
            pl.BlockSpec((_TB,), lambda i: (i,)),
        ],
        out_specs=pl.BlockSpec((1, 1), lambda i: (0, 0)),
        out_shape=jax.ShapeDtypeStruct((1, 1), jnp.float32),
    )(context_embeddings, principals, actT, rowids)
    return out[0, 0]


# speedup vs baseline: 1.3370x; 1.3370x over previous
import jax
import jax.numpy as jnp
from jax import lax
from jax.experimental import pallas as pl
from jax.experimental.pallas import tpu as pltpu

_EPSILON = 1e-06
_SOFT_MARGIN = 0.5
_HARD_MARGIN = 0.1
_NUM_NEG = 8
_B = 16
_D = 128
_T = 4096


_G = 1
_TB = _T // _G


def _loss_kernel(ctx_ref, prin_ref, actT_ref, rowid_ref, out_ref):
    logits = lax.dot_general(ctx_ref[...], actT_ref[...],
                             dimension_numbers=(((1,), (1,)), ((), ())),
                             preferred_element_type=jnp.float32)
    logits = logits * (1.0 / (_D ** 0.5))
    sc = jax.nn.sigmoid(logits)
    s = -jnp.log(_EPSILON + 1.0 - sc)

    row = lax.broadcasted_iota(jnp.int32, (_B, _TB), 0)
    rid = rowid_ref[...].reshape(1, _TB)
    off = (row - rid) & (_B - 1)

    pos = off == 0
    neg = (off >= 1) & (off <= _NUM_NEG)

    s_pos = jnp.sum(jnp.where(pos, s, 0.0), axis=0, keepdims=True)

    prin_row = jnp.transpose(prin_ref[...])
    prin_q = jnp.sum(jnp.where(pos, prin_row, 0), axis=0, keepdims=True)
    w = jnp.where(neg & (prin_row != prin_q), 1.0, 0.0)

    x = s - s_pos + _HARD_MARGIN
    quad = x * x / (2.0 * _SOFT_MARGIN)
    lin = x - _SOFT_MARGIN / 2.0
    h = jnp.where(x <= 0.0, 0.0, jnp.where(x < _SOFT_MARGIN, quad, lin))

    num = jnp.sum(h * w, axis=0, keepdims=True)
    den = jnp.sum(w, axis=0, keepdims=True) + _EPSILON
    per_query = num / den
    partial = jnp.sum(per_query) * (1.0 / (_T + _EPSILON))

    @pl.when(pl.program_id(0) == 0)
    def _init():
        out_ref[0] = partial

    @pl.when(pl.program_id(0) > 0)
    def _acc():
        out_ref[0] += partial


def kernel(context_embeddings, principals, action_flat, action_rowids):
    principals = jnp.squeeze(principals).reshape(1, _B)
    actT = action_flat
    rowids = action_rowids
    out = pl.pallas_call(
        _loss_kernel,
        grid=(_G,),
        in_specs=[
            pl.BlockSpec((_B, _D), lambda i: (0, 0)),
            pl.BlockSpec((1, _B), lambda i: (0, 0)),
            pl.BlockSpec((_TB, _D), lambda i: (i, 0)),
            pl.BlockSpec((_TB,), lambda i: (i,)),
        ],
        out_specs=pl.BlockSpec(memory_space=pltpu.MemorySpace.SMEM),
        out_shape=jax.ShapeDtypeStruct((1,), jnp.float32),
    )(context_embeddings, principals, actT, rowids)
    return out[0]


# speedup vs baseline: 1.3643x; 1.0204x over previous
import jax
import jax.numpy as jnp
from jax import lax
from jax.experimental import pallas as pl

_EPSILON = 1e-06
_SOFT_MARGIN = 0.5
_HARD_MARGIN = 0.1
_NUM_NEG = 8
_B = 16
_D = 128
_T = 4096


_G = 1
_TB = _T // _G


def _loss_kernel(ctx_ref, prin_ref, actT_ref, rowid_ref, out_ref):
    logits = lax.dot_general(ctx_ref[...], actT_ref[...],
                             dimension_numbers=(((1,), (1,)), ((), ())),
                             preferred_element_type=jnp.float32)
    logits = logits * (1.0 / (_D ** 0.5))
    sc = jax.nn.sigmoid(logits)
    s = -jnp.log(_EPSILON + 1.0 - sc)

    row = lax.broadcasted_iota(jnp.int32, (_B, _TB), 0)
    rid = rowid_ref[...].reshape(1, _TB)
    off = (row - rid) & (_B - 1)

    pos = off == 0
    neg = (off >= 1) & (off <= _NUM_NEG)

    s_pos = jnp.sum(jnp.where(pos, s, 0.0), axis=0, keepdims=True)

    prin_row = jnp.transpose(prin_ref[...])
    prin_q = jnp.sum(jnp.where(pos, prin_row, 0), axis=0, keepdims=True)
    w = jnp.where(neg & (prin_row != prin_q), 1.0, 0.0)

    x = s - s_pos + _HARD_MARGIN
    quad = x * x / (2.0 * _SOFT_MARGIN)
    lin = x - _SOFT_MARGIN / 2.0
    h = jnp.where(x <= 0.0, 0.0, jnp.where(x < _SOFT_MARGIN, quad, lin))

    num = jnp.sum(h * w, axis=0, keepdims=True)
    den = jnp.sum(w, axis=0, keepdims=True) + _EPSILON
    per_query = num / den
    partial = jnp.sum(per_query, axis=1, keepdims=True) * (1.0 / (_T + _EPSILON))

    @pl.when(pl.program_id(0) == 0)
    def _init():
        out_ref[...] = partial

    @pl.when(pl.program_id(0) > 0)
    def _acc():
        out_ref[...] += partial


def kernel(context_embeddings, principals, action_flat, action_rowids):
    principals = jnp.squeeze(principals).reshape(1, _B)
    actT = action_flat
    rowids = action_rowids
    out = pl.pallas_call(
        _loss_kernel,
        grid=(_G,),
        in_specs=[
            pl.BlockSpec((_B, _D), lambda i: (0, 0)),
            pl.BlockSpec((1, _B), lambda i: (0, 0)),
            pl.BlockSpec((_TB, _D), lambda i: (i, 0)),
            pl.BlockSpec((_TB,), lambda i: (i,)),
        ],
        out_specs=pl.BlockSpec((1, 1), lambda i: (0, 0)),
        out_shape=jax.ShapeDtypeStruct((1, 1), jnp.float32),
    )(context_embeddings, principals, actT, rowids)
    return out[0, 0]
